# baseline (speedup 1.0000x reference)
import jax
import jax.numpy as jnp
from jax.experimental import pallas as pl

EPS = 1e-6
BN_EPS = 1e-5


def _eig_apply(X, f):
    w, V = jnp.linalg.eigh(X)
    return jnp.einsum('...ij,...j,...kj->...ik', V, f(w), V)


def _copy_kernel(x_ref, o_ref):
    o_ref[...] = x_ref[...]


def kernel(X, running_mean, running_var, mean, std):
    B, C, n, _ = X.shape
    bm = X.mean(axis=0, keepdims=True)
    w, V = jnp.linalg.eigh(bm)
    wc = jnp.clip(w, EPS)
    bm_sq = jnp.einsum('...ij,...j,...kj->...ik', V, jnp.sqrt(wc), V)
    bm_isq = jnp.einsum('...ij,...j,...kj->...ik', V, 1.0 / jnp.sqrt(wc), V)
    XT = _eig_apply(bm_isq @ X @ bm_isq, lambda t: jnp.log(jnp.clip(t, EPS)))
    GT = XT.mean(axis=0, keepdims=True)
    batch_mean = bm_sq @ _eig_apply(GT, jnp.exp) @ bm_sq
    nrm2 = jnp.sum(XT * XT, axis=(-2, -1))
    batch_var = nrm2.mean(axis=0)[None, :, None] - jnp.sum(GT * GT, axis=(-2, -1))[..., None]
    s = std / jnp.sqrt(batch_var + BN_EPS)
    rm_isq = _eig_apply(batch_mean, lambda t: 1.0 / jnp.sqrt(jnp.clip(t, EPS)))
    mean_sq = _eig_apply(mean, lambda t: jnp.sqrt(jnp.clip(t, 0.0)))
    inner = rm_isq @ X @ rm_isq
    wi, Vi = jnp.linalg.eigh(inner)
    pw = jnp.exp(s * jnp.log(jnp.clip(wi, EPS)))
    Xn = mean_sq @ jnp.einsum('...ij,...j,...kj->...ik', Vi, pw, Vi) @ mean_sq

    out = pl.pallas_call(
        _copy_kernel,
        out_shape=jax.ShapeDtypeStruct(Xn.shape, Xn.dtype),
        grid=(B // 64,),
        in_specs=[pl.BlockSpec((64, C, n, n), lambda i: (i, 0, 0, 0))],
        out_specs=pl.BlockSpec((64, C, n, n), lambda i: (i, 0, 0, 0)),
    )(Xn)
    return out


# speedup vs baseline: 30.1921x; 30.1921x over previous
import functools

import jax
import jax.numpy as jnp
from jax.experimental import pallas as pl
from jax.experimental.pallas import tpu as pltpu

BN_EPS = 1e-5
_HI = jax.lax.Precision.HIGHEST
_MED = jax.lax.Precision.HIGHEST

_STAGE1 = ((5.285020928775389, -4.815760163874078),
           (1.548634158945582, -0.2687940701582461),
           (1.5145465331411607, -0.42837789766498846),
           (1.5010459128709432, -0.4947762631854132))
_STAGE2 = ((2.807119527479886, -2.021673969320264),
           (1.5069674507047714, -0.465419664617447))
_M_LOG = 0.6495348781221221
_LOGC = (-3.0764425174023557e-09, 0.9999997197888434, -0.4999989136176829,
         0.3333614697548439, -0.25006171183389037, 0.19920715010350082,
         -0.1653739176796296, 0.15221359134397985, -0.13760423865547136,
         0.05944291330887182, -0.03934401805491741, 0.21629831864437957,
         -0.21520829840879455)
_CORR = (1.0, -0.5, 0.375, -0.3125)
_CHAINC = ((5.965835101773974, -5.545947644777327),
           (1.557857380727064, -0.22748158687265177),
           (1.5218160083942047, -0.39339808140980226),
           (1.5024654524404233, -0.48770506764014476),
           (1.5000277912653754, -0.4998610477922053))
_EXPC = (1.0 / 720, 1.0 / 120, 1.0 / 24, 1.0 / 6, 0.5, 1.0, 1.0)


def _eye(N):
    r = jax.lax.broadcasted_iota(jnp.int32, (N, N), 0)
    c = jax.lax.broadcasted_iota(jnp.int32, (N, N), 1)
    return jnp.where(r == c, 1.0, 0.0).astype(jnp.float32)


def _mm(a, b, prec=None):
    return jnp.dot(a, b, preferred_element_type=jnp.float32, precision=prec)


def _blocksum_col(v, nb, n):
    s = jnp.sum(v.reshape(nb, n, 1), axis=1, keepdims=True)
    return jnp.broadcast_to(s, (nb, n, 1)).reshape(nb * n, 1)


def _sqrt_stage(S, coef, EYE, prec=None):
    P = S
    Y = S
    for (a, b) in coef:
        T = jnp.float32(b) * P + jnp.float32(a) * EYE
        Y = _mm(Y, T, prec)
        P = _mm(_mm(T, T, prec), P, prec)
    E = P - EYE
    H = jnp.float32(_CORR[3]) * E + jnp.float32(_CORR[2]) * EYE
    H = _mm(H, E, prec) + jnp.float32(_CORR[1]) * EYE
    H = _mm(H, E, prec) + jnp.float32(_CORR[0]) * EYE
    return _mm(Y, H, prec)


def _isqrt_sqrt_chain(A, coef, EYE, prec=None):
    P = A
    Y = A
    R = EYE
    for (a, b) in coef:
        T = jnp.float32(b) * P + jnp.float32(a) * EYE
        Y = _mm(Y, T, prec)
        R = _mm(R, T, prec)
        P = _mm(_mm(T, T, prec), P, prec)
    E = P - EYE
    H = jnp.float32(_CORR[3]) * E + jnp.float32(_CORR[2]) * EYE
    H = _mm(H, E, prec) + jnp.float32(_CORR[1]) * EYE
    H = _mm(H, E, prec) + jnp.float32(_CORR[0]) * EYE
    return _mm(Y, H, prec), _mm(R, H, prec)


def _logm_bd(M, EYE, nb, n, prec=None):
    v = jnp.sum(M * M, axis=1, keepdims=True)
    fb = _blocksum_col(v, nb, n)
    b_col = jnp.sqrt(fb)
    P0 = M * (1.0 / b_col)
    S = _sqrt_stage(P0, _STAGE1, EYE, prec)
    S = _sqrt_stage(S, _STAGE2, EYE, prec)
    W = S * jnp.float32(1.0 / _M_LOG) - EYE
    acc = jnp.zeros_like(M)
    for c in reversed(_LOGC):
        acc = _mm(acc, W, prec) + jnp.float32(c) * EYE
    Lmat = 4.0 * (acc + jnp.float32(jnp.log(_M_LOG)) * EYE)
    return Lmat, jnp.log(b_col)


def _expm_bd(A, EYE, j=4, prec=None):
    As = A * jnp.float32(1.0 / (2 ** j))
    acc = jnp.zeros_like(A)
    for c in _EXPC:
        acc = _mm(acc, As, prec) + jnp.float32(c) * EYE
    for _ in range(j):
        acc = _mm(acc, acc, prec)
    return acc


def _bd_build(bd_ref, tile_read, nb, n):
    bd_ref[...] = jnp.zeros((nb * n, nb * n), dtype=jnp.float32)
    for g in range(nb):
        bd_ref[g * n:(g + 1) * n, g * n:(g + 1) * n] = tile_read(g)
    return bd_ref[...]


def _p1_kernel(x_ref, o_ref):
    o_ref[...] = jnp.sum(x_ref[...], axis=0, keepdims=True)[None]


def _s1_kernel(bm_ref, sq_ref, isq_ref, bd_ref):
    C, n = bm_ref.shape[0], bm_ref.shape[1]
    N = C * n
    EYE = _eye(N)
    A = _bd_build(bd_ref, lambda g: bm_ref[g], C, n)
    v = jnp.sum(A * A, axis=1, keepdims=True)
    fb = _blocksum_col(v, C, n)
    b_col = jnp.sqrt(fb)
    P0 = A * (1.0 / b_col)
    Ysq, Risq = _isqrt_sqrt_chain(P0, _CHAINC, EYE, _HI)
    bm_sq = Ysq * jnp.sqrt(b_col)
    bm_isq = Risq * (1.0 / jnp.sqrt(b_col))
    for g in range(C):
        sq_ref[g] = bm_sq[g * n:(g + 1) * n, g * n:(g + 1) * n]
        isq_ref[g] = bm_isq[g * n:(g + 1) * n, g * n:(g + 1) * n]


def _p2_kernel(x_ref, w_ref, xt_ref, nrm_ref, bd_ref, *, nb, n):
    N = nb * n
    EYE = _eye(N)

    @pl.when(pl.program_id(2) == 0)
    def _init():
        xt_ref[...] = jnp.zeros_like(xt_ref)
        nrm_ref[...] = jnp.zeros_like(nrm_ref)

    X = _bd_build(bd_ref, lambda g: x_ref[g, 0], nb, n)
    W = w_ref[0]
    M = _mm(_mm(W, X), W)
    Lmat, logb = _logm_bd(M, EYE, nb, n)
    XT = Lmat + EYE * logb
    xt_ref[0, 0] += XT
    t = XT * XT
    nrm_ref[0, 0] += jnp.sum(t.reshape(n, nb, N), axis=0)


def _s2_kernel(gt_ref, bmsq_ref, mean_ref, rmisq_ref, msq_ref, bd_ref):
    C, n = gt_ref.shape[0], gt_ref.shape[1]
    N = C * n
    EYE = _eye(N)
    GT = _bd_build(bd_ref, lambda g: gt_ref[g], C, n)
    Egt = _expm_bd(GT, EYE, 4, _HI)
    Bsq = _bd_build(bd_ref, lambda g: bmsq_ref[g], C, n)
    batch_mean = _mm(_mm(Bsq, Egt, _HI), Bsq, _HI)
    v = jnp.sum(batch_mean * batch_mean, axis=1, keepdims=True)
    b_col = jnp.sqrt(_blocksum_col(v, C, n))
    P0 = batch_mean * (1.0 / b_col)
    _, Risq = _isqrt_sqrt_chain(P0, _CHAINC, EYE, _HI)
    rm_isq = Risq * (1.0 / jnp.sqrt(b_col))

    Mn = _bd_build(bd_ref, lambda g: mean_ref[g], C, n)
    v2 = jnp.sum(Mn * Mn, axis=1, keepdims=True)
    b2 = jnp.sqrt(_blocksum_col(v2, C, n))
    Q0 = Mn * (1.0 / b2)
    Ysq, _ = _isqrt_sqrt_chain(Q0, _CHAINC, EYE, _HI)
    mean_sq = Ysq * jnp.sqrt(b2)
    for g in range(C):
        rmisq_ref[g] = rm_isq[g * n:(g + 1) * n, g * n:(g + 1) * n]
        msq_ref[g] = mean_sq[g * n:(g + 1) * n, g * n:(g + 1) * n]


def _p3_kernel(x_ref, wr_ref, vm_ref, s_ref, o_ref, bd_ref, *, nb, n):
    N = nb * n
    EYE = _eye(N)
    X = _bd_build(bd_ref, lambda g: x_ref[g, 0], nb, n)
    Wr = wr_ref[0]
    inner = _mm(_mm(Wr, X, _MED), Wr, _MED)
    Lmat, logb = _logm_bd(inner, EYE, nb, n, _MED)
    s = s_ref[0, 0, 0]
    d = jnp.sum(Lmat * EYE, axis=1, keepdims=True)
    mu = _blocksum_col(d, nb, n) * jnp.float32(1.0 / n)
    Lc = Lmat - EYE * mu
    fac = jnp.exp(s * (mu + logb))
    E = _expm_bd(s * Lc, EYE, 4, _MED)
    V = vm_ref[0]
    out = _mm(_mm(V, E, _MED), V, _MED) * fac
    for g in range(nb):
        o_ref[g, 0] = out[g * n:(g + 1) * n, g * n:(g + 1) * n]


def kernel(X, running_mean, running_var, mean, std):
    del running_mean, running_var
    B, C, n, _ = X.shape
    nb = 8
    N = nb * n
    ngroups = B // nb
    PAR = 16 if ngroups % 16 == 0 else (8 if ngroups % 8 == 0 else 1)
    IN = ngroups // PAR
    f32 = jnp.float32

    F = C * n * n
    RB = 8 if B % (8 * 512) == 0 else 1
    Bblk = B // RB
    partials = pl.pallas_call(
        _p1_kernel,
        grid=(RB,),
        in_specs=[pl.BlockSpec((Bblk, F), lambda p: (p, 0))],
        out_specs=pl.BlockSpec((1, 1, F), lambda p: (p, 0, 0)),
        out_shape=jax.ShapeDtypeStruct((RB, 1, F), f32),
        compiler_params=pltpu.CompilerParams(
            dimension_semantics=("parallel",)),
    )(X.reshape(B, F))
    bm = (partials.sum(axis=(0, 1)) / f32(B)).reshape(C, n, n)

    bm_sq, bm_isq = pl.pallas_call(
        _s1_kernel,
        grid=(1,),
        in_specs=[pl.BlockSpec((C, n, n), lambda i: (0, 0, 0))],
        out_specs=[pl.BlockSpec((C, n, n), lambda i: (0, 0, 0))] * 2,
        out_shape=[jax.ShapeDtypeStruct((C, n, n), f32)] * 2,
        scratch_shapes=[pltpu.VMEM((C * n, C * n), f32)],
    )(bm)

    W_bd = jnp.einsum('gh,cij->cgihj', jnp.eye(nb, dtype=f32),
                      bm_isq).reshape(C, N, N)
    xt_p, nrm_p = pl.pallas_call(
        functools.partial(_p2_kernel, nb=nb, n=n),
        grid=(PAR, C, IN),
        in_specs=[
            pl.BlockSpec((nb, 1, n, n), lambda p, c, i: (p * IN + i, c, 0, 0)),
            pl.BlockSpec((1, N, N), lambda p, c, i: (c, 0, 0)),
        ],
        out_specs=[
            pl.BlockSpec((1, 1, N, N), lambda p, c, i: (p, c, 0, 0)),
            pl.BlockSpec((1, 1, nb, N), lambda p, c, i: (p, c, 0, 0)),
        ],
        out_shape=[jax.ShapeDtypeStruct((PAR, C, N, N), f32),
                   jax.ShapeDtypeStruct((PAR, C, nb, N), f32)],
        scratch_shapes=[pltpu.VMEM((N, N), f32)],
        compiler_params=pltpu.CompilerParams(
            dimension_semantics=("parallel", "arbitrary", "arbitrary")),
    )(X, W_bd)
    xt_bd = xt_p.sum(axis=0)
    GT = jnp.einsum('cgigj->cij',
                    xt_bd.reshape(C, nb, n, nb, n)) / f32(B)
    nrm_sum = nrm_p.sum(axis=(0, 2, 3))
    batch_var = nrm_sum / f32(B) - jnp.sum(GT * GT, axis=(-2, -1))
    s_ch = (std[:, 0] / jnp.sqrt(batch_var + f32(BN_EPS))).astype(f32)

    rm_isq, mean_sq = pl.pallas_call(
        _s2_kernel,
        grid=(1,),
        in_specs=[pl.BlockSpec((C, n, n), lambda i: (0, 0, 0))] * 3,
        out_specs=[pl.BlockSpec((C, n, n), lambda i: (0, 0, 0))] * 2,
        out_shape=[jax.ShapeDtypeStruct((C, n, n), f32)] * 2,
        scratch_shapes=[pltpu.VMEM((C * n, C * n), f32)],
    )(GT, bm_sq, mean)

    Wr_bd = jnp.einsum('gh,cij->cgihj', jnp.eye(nb, dtype=f32),
                       rm_isq).reshape(C, N, N)
    Vm_bd = jnp.einsum('gh,cij->cgihj', jnp.eye(nb, dtype=f32),
                       mean_sq).reshape(C, N, N)
    s_arr = jnp.broadcast_to(s_ch[:, None, None], (C, 8, 128)).astype(f32)
    Xn = pl.pallas_call(
        functools.partial(_p3_kernel, nb=nb, n=n),
        grid=(PAR, C, IN),
        in_specs=[
            pl.BlockSpec((nb, 1, n, n), lambda p, c, i: (p * IN + i, c, 0, 0)),
            pl.BlockSpec((1, N, N), lambda p, c, i: (c, 0, 0)),
            pl.BlockSpec((1, N, N), lambda p, c, i: (c, 0, 0)),
            pl.BlockSpec((1, 8, 128), lambda p, c, i: (c, 0, 0)),
        ],
        out_specs=pl.BlockSpec((nb, 1, n, n), lambda p, c, i: (p * IN + i, c, 0, 0)),
        out_shape=jax.ShapeDtypeStruct((B, C, n, n), f32),
        scratch_shapes=[pltpu.VMEM((N, N), f32)],
        compiler_params=pltpu.CompilerParams(
            dimension_semantics=("parallel", "arbitrary", "arbitrary")),
    )(X, Wr_bd, Vm_bd, s_arr)
    return Xn


# speedup vs baseline: 55.2048x; 1.8285x over previous
import functools

import jax
import jax.numpy as jnp
from jax.experimental import pallas as pl
from jax.experimental.pallas import tpu as pltpu

BN_EPS = 1e-5
_HI = jax.lax.Precision.HIGHEST

_STAGE1 = ((5.285020928775389, -4.815760163874078),
           (1.548634158945582, -0.2687940701582461),
           (1.5145465331411607, -0.42837789766498846),
           (1.5010459128709432, -0.4947762631854132))
_STAGE2 = ((2.807119527479886, -2.021673969320264),
           (1.5069674507047714, -0.465419664617447))
_M_LOG = 0.6495348781221221
_LOGC = (-3.0764425174023557e-09, 0.9999997197888434, -0.4999989136176829,
         0.3333614697548439, -0.25006171183389037, 0.19920715010350082,
         -0.1653739176796296, 0.15221359134397985, -0.13760423865547136,
         0.05944291330887182, -0.03934401805491741, 0.21629831864437957,
         -0.21520829840879455)
_CORR = (1.0, -0.5, 0.375, -0.3125)
_CHAINC = ((5.965835101773974, -5.545947644777327),
           (1.557857380727064, -0.22748158687265177),
           (1.5218160083942047, -0.39339808140980226),
           (1.5024654524404233, -0.48770506764014476),
           (1.5000277912653754, -0.4998610477922053))
_EXPC = (1.0 / 720, 1.0 / 120, 1.0 / 24, 1.0 / 6, 0.5, 1.0, 1.0)


def _eye(N):
    r = jax.lax.broadcasted_iota(jnp.int32, (N, N), 0)
    c = jax.lax.broadcasted_iota(jnp.int32, (N, N), 1)
    return jnp.where(r == c, 1.0, 0.0).astype(jnp.float32)


def _mm(a, b, prec=None):
    return jnp.dot(a, b, preferred_element_type=jnp.float32, precision=prec)


def _blocksum_col(v, nb, n):
    s = jnp.sum(v.reshape(nb, n, 1), axis=1, keepdims=True)
    return jnp.broadcast_to(s, (nb, n, 1)).reshape(nb * n, 1)


def _sqrt_stage(S, coef, EYE, prec=None):
    P = S
    Y = S
    for (a, b) in coef:
        T = jnp.float32(b) * P + jnp.float32(a) * EYE
        Y = _mm(Y, T, prec)
        P = _mm(_mm(T, T, prec), P, prec)
    E = P - EYE
    H = jnp.float32(_CORR[3]) * E + jnp.float32(_CORR[2]) * EYE
    H = _mm(H, E, prec) + jnp.float32(_CORR[1]) * EYE
    H = _mm(H, E, prec) + jnp.float32(_CORR[0]) * EYE
    return _mm(Y, H, prec)


def _isqrt_sqrt_chain(A, coef, EYE, prec=None):
    P = A
    Y = A
    R = EYE
    for (a, b) in coef:
        T = jnp.float32(b) * P + jnp.float32(a) * EYE
        Y = _mm(Y, T, prec)
        R = _mm(R, T, prec)
        P = _mm(_mm(T, T, prec), P, prec)
    E = P - EYE
    H = jnp.float32(_CORR[3]) * E + jnp.float32(_CORR[2]) * EYE
    H = _mm(H, E, prec) + jnp.float32(_CORR[1]) * EYE
    H = _mm(H, E, prec) + jnp.float32(_CORR[0]) * EYE
    return _mm(Y, H, prec), _mm(R, H, prec)


def _logm_bd(M, EYE, nb, n, prec=None):
    v = jnp.sum(M * M, axis=1, keepdims=True)
    fb = _blocksum_col(v, nb, n)
    b_col = jnp.sqrt(fb)
    P0 = M * (1.0 / b_col)
    S = _sqrt_stage(P0, _STAGE1, EYE, prec)
    S = _sqrt_stage(S, _STAGE2, EYE, prec)
    W = S * jnp.float32(1.0 / _M_LOG) - EYE
    W2 = _mm(W, W, prec)
    W3 = _mm(W2, W, prec)
    W4 = _mm(W2, W2, prec)
    c = [jnp.float32(x) for x in _LOGC]

    def blk(i):
        return c[i] * EYE + c[i + 1] * W + c[i + 2] * W2 + c[i + 3] * W3
    acc = blk(8) + c[12] * W4
    acc = _mm(acc, W4, prec) + blk(4)
    acc = _mm(acc, W4, prec) + blk(0)
    Lmat = 4.0 * (acc + jnp.float32(jnp.log(_M_LOG)) * EYE)
    return Lmat, jnp.log(b_col)


def _expm_bd(A, EYE, j=4, prec=None):
    As = A * jnp.float32(1.0 / (2 ** j))
    U = _mm(As, As, prec)
    V = _mm(U, As, prec)
    lo = EYE + As + jnp.float32(0.5) * U + jnp.float32(1.0 / 6) * V
    hi = (jnp.float32(1.0 / 24) * As + jnp.float32(1.0 / 120) * U
          + jnp.float32(1.0 / 720) * V)
    acc = _mm(V, hi, prec) + lo
    for _ in range(j):
        acc = _mm(acc, acc, prec)
    return acc


def _bd_build(bd_ref, tile_read, nb, n):
    bd_ref[...] = jnp.zeros((nb * n, nb * n), dtype=jnp.float32)
    for g in range(nb):
        bd_ref[g * n:(g + 1) * n, g * n:(g + 1) * n] = tile_read(g)
    return bd_ref[...]


def _p1_kernel(x_ref, o_ref):
    @pl.when(pl.program_id(1) == 0)
    def _init():
        o_ref[...] = jnp.zeros_like(o_ref)

    o_ref[...] += jnp.sum(x_ref[...], axis=0, keepdims=True)[None]


def _s1_kernel(bm_ref, sq_ref, isq_ref, bd_ref):
    C, n = bm_ref.shape[0], bm_ref.shape[1]
    N = C * n
    EYE = _eye(N)
    A = _bd_build(bd_ref, lambda g: bm_ref[g], C, n)
    v = jnp.sum(A * A, axis=1, keepdims=True)
    fb = _blocksum_col(v, C, n)
    b_col = jnp.sqrt(fb)
    P0 = A * (1.0 / b_col)
    Ysq, Risq = _isqrt_sqrt_chain(P0, _CHAINC, EYE, _HI)
    bm_sq = Ysq * jnp.sqrt(b_col)
    bm_isq = Risq * (1.0 / jnp.sqrt(b_col))
    for g in range(C):
        sq_ref[g] = bm_sq[g * n:(g + 1) * n, g * n:(g + 1) * n]
        isq_ref[g] = bm_isq[g * n:(g + 1) * n, g * n:(g + 1) * n]


def _p2_kernel(x_ref, w_ref, xt_ref, nrm_ref, bd_ref, *, nb, n):
    N = nb * n
    EYE = _eye(N)

    @pl.when(pl.program_id(2) == 0)
    def _init():
        xt_ref[...] = jnp.zeros_like(xt_ref)
        nrm_ref[...] = jnp.zeros_like(nrm_ref)

    X = _bd_build(bd_ref, lambda g: x_ref[g, 0], nb, n)
    W = w_ref[0]
    M = _mm(_mm(W, X), W)
    Lmat, logb = _logm_bd(M, EYE, nb, n)
    XT = Lmat + EYE * logb
    xt_ref[0, 0] += XT
    t = XT * XT
    nrm_ref[0, 0] += jnp.sum(t.reshape(n, nb, N), axis=0)


def _s2_kernel(gt_ref, bmsq_ref, mean_ref, rmisq_ref, msq_ref, bd_ref):
    C, n = gt_ref.shape[0], gt_ref.shape[1]
    N = C * n
    EYE = _eye(N)
    GT = _bd_build(bd_ref, lambda g: gt_ref[g], C, n)
    Egt = _expm_bd(GT, EYE, 4, _HI)
    Bsq = _bd_build(bd_ref, lambda g: bmsq_ref[g], C, n)
    batch_mean = _mm(_mm(Bsq, Egt, _HI), Bsq, _HI)
    v = jnp.sum(batch_mean * batch_mean, axis=1, keepdims=True)
    b_col = jnp.sqrt(_blocksum_col(v, C, n))
    P0 = batch_mean * (1.0 / b_col)
    _, Risq = _isqrt_sqrt_chain(P0, _CHAINC, EYE, _HI)
    rm_isq = Risq * (1.0 / jnp.sqrt(b_col))

    Mn = _bd_build(bd_ref, lambda g: mean_ref[g], C, n)
    v2 = jnp.sum(Mn * Mn, axis=1, keepdims=True)
    b2 = jnp.sqrt(_blocksum_col(v2, C, n))
    Q0 = Mn * (1.0 / b2)
    Ysq, _ = _isqrt_sqrt_chain(Q0, _CHAINC, EYE, _HI)
    mean_sq = Ysq * jnp.sqrt(b2)
    for g in range(C):
        rmisq_ref[g] = rm_isq[g * n:(g + 1) * n, g * n:(g + 1) * n]
        msq_ref[g] = mean_sq[g * n:(g + 1) * n, g * n:(g + 1) * n]


def _p3_kernel(x_ref, wr_ref, vm_ref, s_ref, o_ref, bd_ref, *, nb, n):
    N = nb * n
    EYE = _eye(N)
    X = _bd_build(bd_ref, lambda g: x_ref[g, 0], nb, n)
    Wr = wr_ref[0]
    inner = _mm(_mm(Wr, X), Wr)
    Lmat, logb = _logm_bd(inner, EYE, nb, n)
    s = s_ref[0, 0, 0]
    d = jnp.sum(Lmat * EYE, axis=1, keepdims=True)
    mu = _blocksum_col(d, nb, n) * jnp.float32(1.0 / n)
    Lc = Lmat - EYE * mu
    fac = jnp.exp(s * (mu + logb))
    E = _expm_bd(s * Lc, EYE, 4, _HI)
    V = vm_ref[0]
    out = _mm(_mm(V, E, _HI), V, _HI) * fac
    for g in range(nb):
        o_ref[g, 0] = out[g * n:(g + 1) * n, g * n:(g + 1) * n]


def kernel(X, running_mean, running_var, mean, std):
    del running_mean, running_var
    B, C, n, _ = X.shape
    nb = 8
    N = nb * n
    ngroups = B // nb
    PAR = 2 if ngroups % 2 == 0 else 1
    IN = ngroups // PAR
    f32 = jnp.float32

    F = C * n * n
    Bblk = min(512, B // 2)
    KI = B // (2 * Bblk)
    partials = pl.pallas_call(
        _p1_kernel,
        grid=(2, KI),
        in_specs=[pl.BlockSpec((Bblk, F), lambda p, i: (p * KI + i, 0))],
        out_specs=pl.BlockSpec((1, 1, F), lambda p, i: (p, 0, 0)),
        out_shape=jax.ShapeDtypeStruct((2, 1, F), f32),
        compiler_params=pltpu.CompilerParams(
            dimension_semantics=("parallel", "arbitrary")),
    )(X.reshape(B, F))
    bm = (partials.sum(axis=(0, 1)) / f32(B)).reshape(C, n, n)

    bm_sq, bm_isq = pl.pallas_call(
        _s1_kernel,
        grid=(1,),
        in_specs=[pl.BlockSpec((C, n, n), lambda i: (0, 0, 0))],
        out_specs=[pl.BlockSpec((C, n, n), lambda i: (0, 0, 0))] * 2,
        out_shape=[jax.ShapeDtypeStruct((C, n, n), f32)] * 2,
        scratch_shapes=[pltpu.VMEM((C * n, C * n), f32)],
    )(bm)

    W_bd = jnp.einsum('gh,cij->cgihj', jnp.eye(nb, dtype=f32),
                      bm_isq).reshape(C, N, N)
    xt_p, nrm_p = pl.pallas_call(
        functools.partial(_p2_kernel, nb=nb, n=n),
        grid=(PAR, C, IN),
        in_specs=[
            pl.BlockSpec((nb, 1, n, n), lambda p, c, i: (p * IN + i, c, 0, 0)),
            pl.BlockSpec((1, N, N), lambda p, c, i: (c, 0, 0)),
        ],
        out_specs=[
            pl.BlockSpec((1, 1, N, N), lambda p, c, i: (p, c, 0, 0)),
            pl.BlockSpec((1, 1, nb, N), lambda p, c, i: (p, c, 0, 0)),
        ],
        out_shape=[jax.ShapeDtypeStruct((PAR, C, N, N), f32),
                   jax.ShapeDtypeStruct((PAR, C, nb, N), f32)],
        scratch_shapes=[pltpu.VMEM((N, N), f32)],
        compiler_params=pltpu.CompilerParams(
            dimension_semantics=("parallel", "arbitrary", "arbitrary")),
    )(X, W_bd)
    xt_bd = xt_p.sum(axis=0)
    GT = jnp.einsum('cgigj->cij',
                    xt_bd.reshape(C, nb, n, nb, n)) / f32(B)
    nrm_sum = nrm_p.sum(axis=(0, 2, 3))
    batch_var = nrm_sum / f32(B) - jnp.sum(GT * GT, axis=(-2, -1))
    s_ch = (std[:, 0] / jnp.sqrt(batch_var + f32(BN_EPS))).astype(f32)

    rm_isq, mean_sq = pl.pallas_call(
        _s2_kernel,
        grid=(1,),
        in_specs=[pl.BlockSpec((C, n, n), lambda i: (0, 0, 0))] * 3,
        out_specs=[pl.BlockSpec((C, n, n), lambda i: (0, 0, 0))] * 2,
        out_shape=[jax.ShapeDtypeStruct((C, n, n), f32)] * 2,
        scratch_shapes=[pltpu.VMEM((C * n, C * n), f32)],
    )(GT, bm_sq, mean)

    Wr_bd = jnp.einsum('gh,cij->cgihj', jnp.eye(nb, dtype=f32),
                       rm_isq).reshape(C, N, N)
    Vm_bd = jnp.einsum('gh,cij->cgihj', jnp.eye(nb, dtype=f32),
                       mean_sq).reshape(C, N, N)
    s_arr = jnp.broadcast_to(s_ch[:, None, None], (C, 8, 128)).astype(f32)
    Xn = pl.pallas_call(
        functools.partial(_p3_kernel, nb=nb, n=n),
        grid=(PAR, C, IN),
        in_specs=[
            pl.BlockSpec((nb, 1, n, n), lambda p, c, i: (p * IN + i, c, 0, 0)),
            pl.BlockSpec((1, N, N), lambda p, c, i: (c, 0, 0)),
            pl.BlockSpec((1, N, N), lambda p, c, i: (c, 0, 0)),
            pl.BlockSpec((1, 8, 128), lambda p, c, i: (c, 0, 0)),
        ],
        out_specs=pl.BlockSpec((nb, 1, n, n), lambda p, c, i: (p * IN + i, c, 0, 0)),
        out_shape=jax.ShapeDtypeStruct((B, C, n, n), f32),
        scratch_shapes=[pltpu.VMEM((N, N), f32)],
        compiler_params=pltpu.CompilerParams(
            dimension_semantics=("parallel", "arbitrary", "arbitrary")),
    )(X, Wr_bd, Vm_bd, s_arr)
    return Xn


# speedup vs baseline: 59.3836x; 1.0757x over previous
import functools

import jax
import jax.numpy as jnp
from jax.experimental import pallas as pl
from jax.experimental.pallas import tpu as pltpu

BN_EPS = 1e-5
_HI = jax.lax.Precision.HIGHEST

_STAGE1 = ((5.285020928775389, -4.815760163874078),
           (1.548634158945582, -0.2687940701582461),
           (1.5145465331411607, -0.42837789766498846),
           (1.5010459128709432, -0.4947762631854132))
_STAGE2 = ((2.807119527479886, -2.021673969320264),
           (1.5069674507047714, -0.465419664617447))
_M_LOG = 0.6495348781221221
_LOGC = (-3.0764425174023557e-09, 0.9999997197888434, -0.4999989136176829,
         0.3333614697548439, -0.25006171183389037, 0.19920715010350082,
         -0.1653739176796296, 0.15221359134397985, -0.13760423865547136,
         0.05944291330887182, -0.03934401805491741, 0.21629831864437957,
         -0.21520829840879455)
_CORR = (1.0, -0.5, 0.375, -0.3125)
_CHAINC = ((5.965835101773974, -5.545947644777327),
           (1.557857380727064, -0.22748158687265177),
           (1.5218160083942047, -0.39339808140980226),
           (1.5024654524404233, -0.48770506764014476),
           (1.5000277912653754, -0.4998610477922053))
_EXPC = (1.0 / 720, 1.0 / 120, 1.0 / 24, 1.0 / 6, 0.5, 1.0, 1.0)


def _eye(N):
    r = jax.lax.broadcasted_iota(jnp.int32, (N, N), 0)
    c = jax.lax.broadcasted_iota(jnp.int32, (N, N), 1)
    return jnp.where(r == c, 1.0, 0.0).astype(jnp.float32)


def _mm(a, b, prec=None):
    return jnp.dot(a, b, preferred_element_type=jnp.float32, precision=prec)


def _blocksum_col(v, nb, n):
    s = jnp.sum(v.reshape(nb, n, 1), axis=1, keepdims=True)
    return jnp.broadcast_to(s, (nb, n, 1)).reshape(nb * n, 1)


def _sqrt_stage(S, coef, EYE, prec=None):
    P = S
    Y = S
    for (a, b) in coef:
        T = jnp.float32(b) * P + jnp.float32(a) * EYE
        Y = _mm(Y, T, prec)
        P = _mm(_mm(T, T, prec), P, prec)
    E = P - EYE
    H = jnp.float32(_CORR[3]) * E + jnp.float32(_CORR[2]) * EYE
    H = _mm(H, E, prec) + jnp.float32(_CORR[1]) * EYE
    H = _mm(H, E, prec) + jnp.float32(_CORR[0]) * EYE
    return _mm(Y, H, prec)


def _isqrt_sqrt_chain(A, coef, EYE, prec=None):
    P = A
    Y = A
    R = EYE
    for (a, b) in coef:
        T = jnp.float32(b) * P + jnp.float32(a) * EYE
        Y = _mm(Y, T, prec)
        R = _mm(R, T, prec)
        P = _mm(_mm(T, T, prec), P, prec)
    E = P - EYE
    H = jnp.float32(_CORR[3]) * E + jnp.float32(_CORR[2]) * EYE
    H = _mm(H, E, prec) + jnp.float32(_CORR[1]) * EYE
    H = _mm(H, E, prec) + jnp.float32(_CORR[0]) * EYE
    return _mm(Y, H, prec), _mm(R, H, prec)


def _logm_bd(M, EYE, nb, n, prec=None):
    v = jnp.sum(M * M, axis=1, keepdims=True)
    fb = _blocksum_col(v, nb, n)
    b_col = jnp.sqrt(fb)
    P0 = M * (1.0 / b_col)
    S = _sqrt_stage(P0, _STAGE1, EYE, prec)
    S = _sqrt_stage(S, _STAGE2, EYE, prec)
    W = S * jnp.float32(1.0 / _M_LOG) - EYE
    W2 = _mm(W, W, prec)
    W3 = _mm(W2, W, prec)
    W4 = _mm(W2, W2, prec)
    c = [jnp.float32(x) for x in _LOGC]

    def blk(i):
        return c[i] * EYE + c[i + 1] * W + c[i + 2] * W2 + c[i + 3] * W3
    acc = blk(8) + c[12] * W4
    acc = _mm(acc, W4, prec) + blk(4)
    acc = _mm(acc, W4, prec) + blk(0)
    Lmat = 4.0 * (acc + jnp.float32(jnp.log(_M_LOG)) * EYE)
    return Lmat, jnp.log(b_col)


def _expm_bd(A, EYE, j=4, prec=None):
    As = A * jnp.float32(1.0 / (2 ** j))
    U = _mm(As, As, prec)
    V = _mm(U, As, prec)
    lo = EYE + As + jnp.float32(0.5) * U + jnp.float32(1.0 / 6) * V
    hi = (jnp.float32(1.0 / 24) * As + jnp.float32(1.0 / 120) * U
          + jnp.float32(1.0 / 720) * V)
    acc = _mm(V, hi, prec) + lo
    for _ in range(j):
        acc = _mm(acc, acc, prec)
    return acc


def _bd_build(bd_ref, tile_read, nb, n):
    bd_ref[...] = jnp.zeros((nb * n, nb * n), dtype=jnp.float32)
    for g in range(nb):
        bd_ref[g * n:(g + 1) * n, g * n:(g + 1) * n] = tile_read(g)
    return bd_ref[...]


def _p1_kernel(x_ref, o_ref):
    @pl.when(pl.program_id(1) == 0)
    def _init():
        o_ref[...] = jnp.zeros_like(o_ref)

    o_ref[...] += jnp.sum(x_ref[...], axis=0, keepdims=True)[None]


def _s1_kernel(bm_ref, sq_ref, isq_ref, bd_ref):
    C, n = bm_ref.shape[0], bm_ref.shape[1]
    N = C * n
    EYE = _eye(N)
    A = _bd_build(bd_ref, lambda g: bm_ref[g], C, n)
    v = jnp.sum(A * A, axis=1, keepdims=True)
    fb = _blocksum_col(v, C, n)
    b_col = jnp.sqrt(fb)
    P0 = A * (1.0 / b_col)
    Ysq, Risq = _isqrt_sqrt_chain(P0, _CHAINC, EYE, _HI)
    bm_sq = Ysq * jnp.sqrt(b_col)
    bm_isq = Risq * (1.0 / jnp.sqrt(b_col))
    for g in range(C):
        sq_ref[g] = bm_sq[g * n:(g + 1) * n, g * n:(g + 1) * n]
        isq_ref[g] = bm_isq[g * n:(g + 1) * n, g * n:(g + 1) * n]


def _p2_kernel(x_ref, w_ref, xt_ref, nrm_ref, bd_ref, *, nb, n, gs):
    N = nb * n
    EYE = _eye(N)

    @pl.when(pl.program_id(2) == 0)
    def _init():
        xt_ref[...] = jnp.zeros_like(xt_ref)
        nrm_ref[...] = jnp.zeros_like(nrm_ref)

    W = w_ref[0]
    xt_acc = None
    nrm_acc = None
    for q in range(gs):
        X = _bd_build(bd_ref.at[q], lambda g: x_ref[q * nb + g, 0], nb, n)
        M = _mm(_mm(W, X), W)
        Lmat, logb = _logm_bd(M, EYE, nb, n)
        XT = Lmat + EYE * logb
        t = XT * XT
        nrm = jnp.sum(t.reshape(n, nb, N), axis=0)
        xt_acc = XT if xt_acc is None else xt_acc + XT
        nrm_acc = nrm if nrm_acc is None else nrm_acc + nrm
    xt_ref[0, 0] += xt_acc
    nrm_ref[0, 0] += nrm_acc


def _s2_kernel(gt_ref, bmsq_ref, mean_ref, rmisq_ref, msq_ref, bd_ref):
    C, n = gt_ref.shape[0], gt_ref.shape[1]
    N = C * n
    EYE = _eye(N)
    GT = _bd_build(bd_ref, lambda g: gt_ref[g], C, n)
    Egt = _expm_bd(GT, EYE, 4, _HI)
    Bsq = _bd_build(bd_ref, lambda g: bmsq_ref[g], C, n)
    batch_mean = _mm(_mm(Bsq, Egt, _HI), Bsq, _HI)
    v = jnp.sum(batch_mean * batch_mean, axis=1, keepdims=True)
    b_col = jnp.sqrt(_blocksum_col(v, C, n))
    P0 = batch_mean * (1.0 / b_col)
    _, Risq = _isqrt_sqrt_chain(P0, _CHAINC, EYE, _HI)
    rm_isq = Risq * (1.0 / jnp.sqrt(b_col))

    Mn = _bd_build(bd_ref, lambda g: mean_ref[g], C, n)
    v2 = jnp.sum(Mn * Mn, axis=1, keepdims=True)
    b2 = jnp.sqrt(_blocksum_col(v2, C, n))
    Q0 = Mn * (1.0 / b2)
    Ysq, _ = _isqrt_sqrt_chain(Q0, _CHAINC, EYE, _HI)
    mean_sq = Ysq * jnp.sqrt(b2)
    for g in range(C):
        rmisq_ref[g] = rm_isq[g * n:(g + 1) * n, g * n:(g + 1) * n]
        msq_ref[g] = mean_sq[g * n:(g + 1) * n, g * n:(g + 1) * n]


def _p3_kernel(x_ref, wr_ref, vm_ref, s_ref, o_ref, bd_ref, *, nb, n, gs):
    N = nb * n
    EYE = _eye(N)
    Wr = wr_ref[0]
    V = vm_ref[0]
    s = s_ref[0, 0, 0]
    for q in range(gs):
        X = _bd_build(bd_ref.at[q], lambda g: x_ref[q * nb + g, 0], nb, n)
        inner = _mm(_mm(Wr, X), Wr)
        Lmat, logb = _logm_bd(inner, EYE, nb, n)
        d = jnp.sum(Lmat * EYE, axis=1, keepdims=True)
        mu = _blocksum_col(d, nb, n) * jnp.float32(1.0 / n)
        Lc = Lmat - EYE * mu
        fac = jnp.exp(s * (mu + logb))
        E = _expm_bd(s * Lc, EYE, 4, _HI)
        out = _mm(_mm(V, E, _HI), V, _HI) * fac
        for g in range(nb):
            o_ref[q * nb + g, 0] = out[g * n:(g + 1) * n, g * n:(g + 1) * n]


def kernel(X, running_mean, running_var, mean, std):
    del running_mean, running_var
    B, C, n, _ = X.shape
    nb = 8
    N = nb * n
    ngroups = B // nb
    GS = 4 if ngroups % 8 == 0 else 1
    PAR = 2 if ngroups % (2 * GS) == 0 else 1
    IN = ngroups // (PAR * GS)
    f32 = jnp.float32

    F = C * n * n
    Bblk = min(512, B // 2)
    KI = B // (2 * Bblk)
    partials = pl.pallas_call(
        _p1_kernel,
        grid=(2, KI),
        in_specs=[pl.BlockSpec((Bblk, F), lambda p, i: (p * KI + i, 0))],
        out_specs=pl.BlockSpec((1, 1, F), lambda p, i: (p, 0, 0)),
        out_shape=jax.ShapeDtypeStruct((2, 1, F), f32),
        compiler_params=pltpu.CompilerParams(
            dimension_semantics=("parallel", "arbitrary")),
    )(X.reshape(B, F))
    bm = (partials.sum(axis=(0, 1)) / f32(B)).reshape(C, n, n)

    bm_sq, bm_isq = pl.pallas_call(
        _s1_kernel,
        grid=(1,),
        in_specs=[pl.BlockSpec((C, n, n), lambda i: (0, 0, 0))],
        out_specs=[pl.BlockSpec((C, n, n), lambda i: (0, 0, 0))] * 2,
        out_shape=[jax.ShapeDtypeStruct((C, n, n), f32)] * 2,
        scratch_shapes=[pltpu.VMEM((C * n, C * n), f32)],
    )(bm)

    W_bd = jnp.einsum('gh,cij->cgihj', jnp.eye(nb, dtype=f32),
                      bm_isq).reshape(C, N, N)
    xt_p, nrm_p = pl.pallas_call(
        functools.partial(_p2_kernel, nb=nb, n=n, gs=GS),
        grid=(PAR, C, IN),
        in_specs=[
            pl.BlockSpec((GS * nb, 1, n, n),
                         lambda p, c, i: (p * IN + i, c, 0, 0)),
            pl.BlockSpec((1, N, N), lambda p, c, i: (c, 0, 0)),
        ],
        out_specs=[
            pl.BlockSpec((1, 1, N, N), lambda p, c, i: (p, c, 0, 0)),
            pl.BlockSpec((1, 1, nb, N), lambda p, c, i: (p, c, 0, 0)),
        ],
        out_shape=[jax.ShapeDtypeStruct((PAR, C, N, N), f32),
                   jax.ShapeDtypeStruct((PAR, C, nb, N), f32)],
        scratch_shapes=[pltpu.VMEM((GS, N, N), f32)],
        compiler_params=pltpu.CompilerParams(
            dimension_semantics=("parallel", "arbitrary", "arbitrary")),
    )(X, W_bd)
    xt_bd = xt_p.sum(axis=0)
    GT = jnp.einsum('cgigj->cij',
                    xt_bd.reshape(C, nb, n, nb, n)) / f32(B)
    nrm_sum = nrm_p.sum(axis=(0, 2, 3))
    batch_var = nrm_sum / f32(B) - jnp.sum(GT * GT, axis=(-2, -1))
    s_ch = (std[:, 0] / jnp.sqrt(batch_var + f32(BN_EPS))).astype(f32)

    rm_isq, mean_sq = pl.pallas_call(
        _s2_kernel,
        grid=(1,),
        in_specs=[pl.BlockSpec((C, n, n), lambda i: (0, 0, 0))] * 3,
        out_specs=[pl.BlockSpec((C, n, n), lambda i: (0, 0, 0))] * 2,
        out_shape=[jax.ShapeDtypeStruct((C, n, n), f32)] * 2,
        scratch_shapes=[pltpu.VMEM((C * n, C * n), f32)],
    )(GT, bm_sq, mean)

    Wr_bd = jnp.einsum('gh,cij->cgihj', jnp.eye(nb, dtype=f32),
                       rm_isq).reshape(C, N, N)
    Vm_bd = jnp.einsum('gh,cij->cgihj', jnp.eye(nb, dtype=f32),
                       mean_sq).reshape(C, N, N)
    s_arr = jnp.broadcast_to(s_ch[:, None, None], (C, 8, 128)).astype(f32)
    Xn = pl.pallas_call(
        functools.partial(_p3_kernel, nb=nb, n=n, gs=GS),
        grid=(PAR, C, IN),
        in_specs=[
            pl.BlockSpec((GS * nb, 1, n, n),
                         lambda p, c, i: (p * IN + i, c, 0, 0)),
            pl.BlockSpec((1, N, N), lambda p, c, i: (c, 0, 0)),
            pl.BlockSpec((1, N, N), lambda p, c, i: (c, 0, 0)),
            pl.BlockSpec((1, 8, 128), lambda p, c, i: (c, 0, 0)),
        ],
        out_specs=pl.BlockSpec((GS * nb, 1, n, n),
                               lambda p, c, i: (p * IN + i, c, 0, 0)),
        out_shape=jax.ShapeDtypeStruct((B, C, n, n), f32),
        scratch_shapes=[pltpu.VMEM((GS, N, N), f32)],
        compiler_params=pltpu.CompilerParams(
            dimension_semantics=("parallel", "arbitrary", "arbitrary")),
    )(X, Wr_bd, Vm_bd, s_arr)
    return Xn
